# Initial kernel scaffold; baseline (speedup 1.0000x reference)
import jax
import jax.numpy as jnp
from jax.experimental import pallas as pl


def kernel(ctx, g, b, w, wb):
    raise NotImplementedError("write your pallas kernel here")


# speedup vs baseline: 2.4072x; 2.4072x over previous
import functools

import jax
import jax.numpy as jnp
from jax.experimental import pallas as pl
from jax.experimental.pallas import tpu as pltpu

_VMEM_LIMIT = 48 * 1024 * 1024


def _prenorm_pool_kernel(ctx_ref, g_ref, b_ref, w_ref, wb_ref, o_ref, *, eps,
                         seq):
    x = ctx_ref[...]
    mu = jnp.mean(x, axis=-1, keepdims=True)
    c = x - mu
    var = jnp.mean(c * c, axis=-1, keepdims=True)
    r = jax.lax.rsqrt(var + eps)
    s = jnp.sum(c * r, axis=1)
    pooled = s * (1.0 / seq) * g_ref[...] + b_ref[...]
    y = jnp.dot(pooled, w_ref[...], preferred_element_type=jnp.float32)
    o_ref[0] = y + wb_ref[...]


def kernel(ctx, g, b, w, wb):
    bsz, seq, din = ctx.shape
    dout = w.shape[-1]
    tb = 4
    grid = (bsz // tb,)

    out = pl.pallas_call(
        functools.partial(_prenorm_pool_kernel, eps=1e-5, seq=seq),
        out_shape=jax.ShapeDtypeStruct((bsz // tb, tb, dout), jnp.float32),
        grid=grid,
        in_specs=[
            pl.BlockSpec((tb, seq, din), lambda i: (i, 0, 0)),
            pl.BlockSpec((1, din), lambda i: (0, 0)),
            pl.BlockSpec((1, din), lambda i: (0, 0)),
            pl.BlockSpec((din, dout), lambda i: (0, 0)),
            pl.BlockSpec((1, dout), lambda i: (0, 0)),
        ],
        out_specs=pl.BlockSpec((1, tb, dout), lambda i: (i, 0, 0)),
        compiler_params=pltpu.CompilerParams(
            dimension_semantics=("parallel",),
            vmem_limit_bytes=_VMEM_LIMIT),
    )(ctx, g.reshape(1, din), b.reshape(1, din), w, wb.reshape(1, dout))
    return out.reshape(bsz, 1, dout)
